# Initial kernel scaffold; baseline (speedup 1.0000x reference)
import jax
import jax.numpy as jnp
from jax import lax
from jax.experimental import pallas as pl
from jax.experimental.pallas import tpu as pltpu
from jax.experimental.pallas import tpu_sc as plsc

N_NODES = 10000
N_EDGES = 320000
D_FEAT = 128

NC = 2
NS = 16
NW = NC * NS
E_PER_W = N_EDGES // NW
B = 80
NCH = E_PER_W // B
NBUF = 5
N_PAD = 10240
ZROWS = N_PAD // NS


def _sc_body(nf_hbm, src_hbm, dst_hbm, out_hbm, src_v, dst_v, rows_v, acc_sh, sems):
    cid = lax.axis_index("c")
    sid = lax.axis_index("s")
    wid = cid * NS + sid

    pltpu.sync_copy(src_hbm.at[wid], src_v)
    pltpu.sync_copy(dst_hbm.at[wid], dst_v)

    zero16 = jnp.zeros((16,), jnp.float32)

    def _zrow(r, carry):
        for k in range(D_FEAT // 16):
            rows_v[0, r, pl.ds(k * 16, 16)] = zero16
        return carry

    lax.fori_loop(0, B, _zrow, 0, unroll=4)
    for t in range(ZROWS // B):
        pltpu.sync_copy(rows_v.at[0], acc_sh.at[pl.ds(sid * ZROWS + t * B, B)])

    plsc.subcore_barrier()

    for b in range(NBUF):
        pltpu.async_copy(nf_hbm.at[src_v.at[b]], rows_v.at[b], sems.at[b])

    def _outer(i, carry):
        j0 = i * NBUF
        for b in range(NBUF):
            j = j0 + b
            pltpu.make_async_copy(
                nf_hbm.at[src_v.at[j]], rows_v.at[b], sems.at[b]
            ).wait()
            pltpu.sync_copy(rows_v.at[b], acc_sh.at[dst_v.at[j]], add=True)

            @pl.when(i < NCH // NBUF - 1)
            def _():
                pltpu.async_copy(
                    nf_hbm.at[src_v.at[j + NBUF]], rows_v.at[b], sems.at[b]
                )

        return carry

    lax.fori_loop(0, NCH // NBUF, _outer, 0)

    plsc.subcore_barrier()
    rows_out = N_NODES // NS
    pltpu.sync_copy(
        acc_sh.at[pl.ds(sid * rows_out, rows_out)],
        out_hbm.at[cid, pl.ds(sid * rows_out, rows_out)],
    )


@jax.jit
def _mp_sc(node_features, src3, dst3):
    mesh = plsc.VectorSubcoreMesh(core_axis_name="c", subcore_axis_name="s")
    return pl.kernel(
        _sc_body,
        out_type=jax.ShapeDtypeStruct((NC, N_NODES, D_FEAT), jnp.float32),
        mesh=mesh,
        scratch_types=[
            pltpu.VMEM((NCH, B), jnp.int32),
            pltpu.VMEM((NCH, B), jnp.int32),
            pltpu.VMEM((NBUF, B, D_FEAT), jnp.float32),
            pltpu.VMEM_SHARED((N_PAD, D_FEAT), jnp.float32),
            pltpu.SemaphoreType.DMA((NBUF,)),
        ],
    )(node_features, src3, dst3)


def _add_body(p_ref, o_ref):
    o_ref[...] = p_ref[0] + p_ref[1]


@jax.jit
def _combine(partials):
    blk = 1250
    return pl.pallas_call(
        _add_body,
        grid=(N_NODES // blk,),
        in_specs=[pl.BlockSpec((NC, blk, D_FEAT), lambda i: (0, i, 0))],
        out_specs=pl.BlockSpec((blk, D_FEAT), lambda i: (i, 0)),
        out_shape=jax.ShapeDtypeStruct((N_NODES, D_FEAT), jnp.float32),
    )(partials)


def kernel(node_features, edge_index):
    src3 = edge_index[0].reshape(NW, NCH, B)
    dst3 = edge_index[1].reshape(NW, NCH, B)
    partials = _mp_sc(node_features, src3, dst3)
    return _combine(partials)


# speedup vs baseline: 6.5363x; 6.5363x over previous
import jax
import jax.numpy as jnp
from jax import lax
from jax.experimental import pallas as pl
from jax.experimental.pallas import tpu as pltpu
from jax.experimental.pallas import tpu_sc as plsc

N_NODES = 10000
N_EDGES = 320000
D_FEAT = 128

NC = 2
NS = 16
E_PER_T = N_EDGES // NS
B = 80
NCH = E_PER_T // B
G = 25
NGRP = NCH // G
NBUF = 5
GI = G // NBUF
HALF = 5120
N_ACC = HALF + NS * 8


def _sc_body(nf_hbm, src_hbm, dst_hbm, out_hbm, gsrc, gdst, zero_v, rows_v, acc_sh, sems, gsems):
    cid = lax.axis_index("c")
    sid = lax.axis_index("s")
    base = cid * HALF
    size = HALF - cid * (2 * HALF - N_NODES)
    garbage = HALF + sid * 8

    zero16 = jnp.zeros((16,), jnp.float32)

    def _zrow(r, carry):
        for k in range(D_FEAT // 16):
            zero_v[r, pl.ds(k * 16, 16)] = zero16
        return carry

    lax.fori_loop(0, 16, _zrow, 0, unroll=4)

    @pl.when(cid == 0)
    def _():
        for t in range(320 // 16):
            pltpu.sync_copy(zero_v, acc_sh.at[pl.ds(sid * 320 + t * 16, 16)])

    @pl.when((cid == 1) & (sid < 15))
    def _():
        for t in range(304 // 16):
            pltpu.sync_copy(zero_v, acc_sh.at[pl.ds(sid * 304 + t * 16, 16)])

    @pl.when((cid == 1) & (sid == 15))
    def _():
        for t in range(320 // 16):
            pltpu.sync_copy(zero_v, acc_sh.at[pl.ds(4560 + t * 16, 16)])

    pltpu.sync_copy(zero_v.at[pl.ds(0, 8)], acc_sh.at[pl.ds(garbage, 8)])

    plsc.subcore_barrier()

    pltpu.sync_copy(src_hbm.at[sid * NGRP], gsrc.at[0])
    pltpu.sync_copy(dst_hbm.at[sid * NGRP], gdst.at[0])

    def _chunk(gb, jl, b, refill):
        pltpu.make_async_copy(
            nf_hbm.at[gsrc.at[gb, jl]], rows_v.at[b], sems.at[b]
        ).wait()
        for k in range(B // 16):
            d = gdst[gb, jl, pl.ds(k * 16, 16)] - base
            ok = (d >= 0) & (d < size)
            gdst[gb, jl, pl.ds(k * 16, 16)] = jnp.where(ok, d, garbage)
        pltpu.sync_copy(rows_v.at[b], acc_sh.at[gdst.at[gb, jl]], add=True)
        if refill is not None:
            @pl.when(refill)
            def _():
                pltpu.async_copy(
                    nf_hbm.at[gsrc.at[gb, jl + NBUF]], rows_v.at[b], sems.at[b]
                )

    def _group(g, carry):
        gb = g % 2

        @pl.when(g > 0)
        def _():
            pltpu.make_async_copy(
                src_hbm.at[sid * NGRP + g], gsrc.at[gb], gsems.at[gb]
            ).wait()
            pltpu.make_async_copy(
                dst_hbm.at[sid * NGRP + g], gdst.at[gb], gsems.at[gb]
            ).wait()

        for b in range(NBUF):
            pltpu.async_copy(nf_hbm.at[gsrc.at[gb, b]], rows_v.at[b], sems.at[b])

        for b in range(NBUF):
            _chunk(gb, b, b, True)

        @pl.when(g + 1 < NGRP)
        def _():
            nb = (g + 1) % 2
            pltpu.async_copy(
                src_hbm.at[sid * NGRP + g + 1], gsrc.at[nb], gsems.at[nb]
            )
            pltpu.async_copy(
                dst_hbm.at[sid * NGRP + g + 1], gdst.at[nb], gsems.at[nb]
            )

        def _round(i, c):
            for b in range(NBUF):
                _chunk(gb, i * NBUF + b, b, i < GI - 1)
            return c

        lax.fori_loop(1, GI, _round, 0)
        return carry

    lax.fori_loop(0, NGRP, _group, 0)

    plsc.subcore_barrier()

    @pl.when(cid == 0)
    def _():
        pltpu.sync_copy(
            acc_sh.at[pl.ds(sid * 320, 320)],
            out_hbm.at[pl.ds(sid * 320, 320)],
        )

    @pl.when((cid == 1) & (sid < 15))
    def _():
        pltpu.sync_copy(
            acc_sh.at[pl.ds(sid * 304, 304)],
            out_hbm.at[pl.ds(HALF + sid * 304, 304)],
        )

    @pl.when((cid == 1) & (sid == 15))
    def _():
        pltpu.sync_copy(
            acc_sh.at[pl.ds(4560, 320)],
            out_hbm.at[pl.ds(HALF + 4560, 320)],
        )


def _mp_sc(node_features, src3, dst3):
    mesh = plsc.VectorSubcoreMesh(core_axis_name="c", subcore_axis_name="s")
    return pl.kernel(
        _sc_body,
        out_type=jax.ShapeDtypeStruct((N_NODES, D_FEAT), jnp.float32),
        mesh=mesh,
        scratch_types=[
            pltpu.VMEM((2, G, B), jnp.int32),
            pltpu.VMEM((2, G, B), jnp.int32),
            pltpu.VMEM((16, D_FEAT), jnp.float32),
            pltpu.VMEM((NBUF, B, D_FEAT), jnp.float32),
            pltpu.VMEM_SHARED((N_ACC, D_FEAT), jnp.float32),
            pltpu.SemaphoreType.DMA((NBUF,)),
            pltpu.SemaphoreType.DMA((2,)),
        ],
    )(node_features, src3, dst3)


def kernel(node_features, edge_index):
    src3 = edge_index[0].reshape(NS * NGRP, G, B)
    dst3 = edge_index[1].reshape(NS * NGRP, G, B)
    return _mp_sc(node_features, src3, dst3)


# speedup vs baseline: 6.6157x; 1.0121x over previous
import jax
import jax.numpy as jnp
from jax import lax
from jax.experimental import pallas as pl
from jax.experimental.pallas import tpu as pltpu
from jax.experimental.pallas import tpu_sc as plsc

N_NODES = 10000
N_EDGES = 320000
D_FEAT = 128

NC = 2
NS = 16
E_PER_T = N_EDGES // NS
B = 80
NCH = E_PER_T // B
G = 25
NGRP = NCH // G
NBUF = 5
GI = G // NBUF
HALF = 5120
N_ACC = HALF + NS * 8


def _sc_body(nf_hbm, src_hbm, dst_hbm, out_hbm, gsrc, gdst, zero_v, rows_v, acc_sh, sems, ssems, gsems):
    cid = lax.axis_index("c")
    sid = lax.axis_index("s")
    base = cid * HALF
    size = HALF - cid * (2 * HALF - N_NODES)
    garbage = HALF + sid * 8

    zero16 = jnp.zeros((16,), jnp.float32)

    def _zrow(r, carry):
        for k in range(D_FEAT // 16):
            zero_v[r, pl.ds(k * 16, 16)] = zero16
        return carry

    lax.fori_loop(0, 16, _zrow, 0, unroll=4)

    @pl.when(cid == 0)
    def _():
        for t in range(320 // 16):
            pltpu.sync_copy(zero_v, acc_sh.at[pl.ds(sid * 320 + t * 16, 16)])

    @pl.when((cid == 1) & (sid < 15))
    def _():
        for t in range(304 // 16):
            pltpu.sync_copy(zero_v, acc_sh.at[pl.ds(sid * 304 + t * 16, 16)])

    @pl.when((cid == 1) & (sid == 15))
    def _():
        for t in range(320 // 16):
            pltpu.sync_copy(zero_v, acc_sh.at[pl.ds(4560 + t * 16, 16)])

    pltpu.sync_copy(zero_v.at[pl.ds(0, 8)], acc_sh.at[pl.ds(garbage, 8)])

    plsc.subcore_barrier()

    pltpu.sync_copy(src_hbm.at[sid * NGRP], gsrc.at[0])
    pltpu.sync_copy(dst_hbm.at[sid * NGRP], gdst.at[0])

    def _scat_wait(gb, b):
        pltpu.make_async_copy(
            rows_v.at[b], acc_sh.at[gdst.at[gb, 0]], ssems.at[b]
        ).wait()

    def _chunk(gb, jl, b, refill):
        pltpu.make_async_copy(
            nf_hbm.at[gsrc.at[gb, jl]], rows_v.at[b], sems.at[b]
        ).wait()
        for k in range(B // 16):
            d = gdst[gb, jl, pl.ds(k * 16, 16)] - base
            ok = (d >= 0) & (d < size)
            gdst[gb, jl, pl.ds(k * 16, 16)] = jnp.where(ok, d, garbage)
        pltpu.async_copy(
            rows_v.at[b], acc_sh.at[gdst.at[gb, jl]], ssems.at[b], add=True
        )
        if refill is not None:
            br = (b - 1) % NBUF

            @pl.when(refill)
            def _():
                _scat_wait(gb, br)
                pltpu.async_copy(
                    nf_hbm.at[gsrc.at[gb, jl - 1 + NBUF]],
                    rows_v.at[br],
                    sems.at[br],
                )

    def _group(g, carry):
        gb = g % 2

        @pl.when(g > 0)
        def _():
            pltpu.make_async_copy(
                src_hbm.at[sid * NGRP + g], gsrc.at[gb], gsems.at[gb]
            ).wait()
            pltpu.make_async_copy(
                dst_hbm.at[sid * NGRP + g], gdst.at[gb], gsems.at[gb]
            ).wait()

        for b in range(NBUF):
            @pl.when(g > 0)
            def _():
                _scat_wait(gb, b)
            pltpu.async_copy(nf_hbm.at[gsrc.at[gb, b]], rows_v.at[b], sems.at[b])

        for b in range(NBUF):
            _chunk(gb, b, b, True if b >= 1 else None)

        @pl.when(g + 1 < NGRP)
        def _():
            nb = (g + 1) % 2
            pltpu.async_copy(
                src_hbm.at[sid * NGRP + g + 1], gsrc.at[nb], gsems.at[nb]
            )
            pltpu.async_copy(
                dst_hbm.at[sid * NGRP + g + 1], gdst.at[nb], gsems.at[nb]
            )

        def _round(i, c):
            for b in range(NBUF):
                jl = i * NBUF + b
                _chunk(gb, jl, b, jl <= G - NBUF)
            return c

        lax.fori_loop(1, GI, _round, 0)
        return carry

    lax.fori_loop(0, NGRP, _group, 0)

    for b in range(NBUF):
        _scat_wait((NGRP - 1) % 2, b)

    plsc.subcore_barrier()

    @pl.when(cid == 0)
    def _():
        pltpu.sync_copy(
            acc_sh.at[pl.ds(sid * 320, 320)],
            out_hbm.at[pl.ds(sid * 320, 320)],
        )

    @pl.when((cid == 1) & (sid < 15))
    def _():
        pltpu.sync_copy(
            acc_sh.at[pl.ds(sid * 304, 304)],
            out_hbm.at[pl.ds(HALF + sid * 304, 304)],
        )

    @pl.when((cid == 1) & (sid == 15))
    def _():
        pltpu.sync_copy(
            acc_sh.at[pl.ds(4560, 320)],
            out_hbm.at[pl.ds(HALF + 4560, 320)],
        )


def _mp_sc(node_features, src3, dst3):
    mesh = plsc.VectorSubcoreMesh(core_axis_name="c", subcore_axis_name="s")
    return pl.kernel(
        _sc_body,
        out_type=jax.ShapeDtypeStruct((N_NODES, D_FEAT), jnp.float32),
        mesh=mesh,
        scratch_types=[
            pltpu.VMEM((2, G, B), jnp.int32),
            pltpu.VMEM((2, G, B), jnp.int32),
            pltpu.VMEM((16, D_FEAT), jnp.float32),
            pltpu.VMEM((NBUF, B, D_FEAT), jnp.float32),
            pltpu.VMEM_SHARED((N_ACC, D_FEAT), jnp.float32),
            pltpu.SemaphoreType.DMA((NBUF,)),
            pltpu.SemaphoreType.DMA((NBUF,)),
            pltpu.SemaphoreType.DMA((2,)),
        ],
    )(node_features, src3, dst3)


def kernel(node_features, edge_index):
    src3 = edge_index[0].reshape(NS * NGRP, G, B)
    dst3 = edge_index[1].reshape(NS * NGRP, G, B)
    return _mp_sc(node_features, src3, dst3)
